# Initial kernel scaffold; baseline (speedup 1.0000x reference)
import jax
import jax.numpy as jnp
from jax.experimental import pallas as pl


def kernel(h_x, h_edge_index, h_batch, t_x, t_edge_index, t_batch, rels, b_edge_index, params):
    raise NotImplementedError("write your pallas kernel here")


# speedup vs baseline: 3.8724x; 3.8724x over previous
import functools

import jax
import jax.numpy as jnp
from jax import lax
from jax.experimental import pallas as pl
from jax.experimental.pallas import tpu as pltpu
from jax.experimental.pallas import tpu_sc as plsc

N = 10000
E = 320000
B = 256
F = 128
H = 128
Hh = 64
R = 86
L = 86

NP = 10240
BR = 1280
NBLK = NP // BR
NC = 2
NS = 16
NW = NC * NS
EP = E // NW
CK = 128
CH = (EP + CK - 1) // CK
EPP = CH * CK
CH = 80
EPP = CH * CK
ROWS_PER_TILE = NP // NS


def _elu(x):
    return jnp.where(x > 0, x, jnp.exp(jnp.minimum(x, 0.0)) - 1.0)


def _ln(x, g, b):
    m = jnp.mean(x, axis=-1, keepdims=True)
    v = jnp.mean((x - m) * (x - m), axis=-1, keepdims=True)
    return (x - m) * jax.lax.rsqrt(v + 1e-5) * g + b


def _prep_body(hx_ref, tx_ref, g_ref, b_ref, wnc_ref, wnb_ref, wd_ref, bd_ref,
               y1_ref, y2_ref, y3_ref, y4_ref, dh_ref, dt_ref):
    g = g_ref[...]
    b = b_ref[...]
    xh = _elu(_ln(hx_ref[...], g, b))
    xt = _elu(_ln(tx_ref[...], g, b))
    wnc = wnc_ref[...]
    wnb = wnb_ref[...]
    wd = wd_ref[...]
    bd = bd_ref[...]
    y1_ref[...] = jnp.dot(xh, wnc, preferred_element_type=jnp.float32)
    y2_ref[...] = jnp.dot(xt, wnc, preferred_element_type=jnp.float32)
    y3_ref[...] = jnp.dot(xh, wnb, preferred_element_type=jnp.float32)
    y4_ref[...] = jnp.dot(xt, wnb, preferred_element_type=jnp.float32)
    dh_ref[...] = jnp.dot(xh, wd, preferred_element_type=jnp.float32) + bd
    dt_ref[...] = jnp.dot(xt, wd, preferred_element_type=jnp.float32) + bd


def _prep(hx, tx, g, b, wnc, wnb, wd, bd):
    row = pl.BlockSpec((BR, F), lambda i: (i, 0))
    full = lambda *shape: pl.BlockSpec(shape, lambda i: (0,) * len(shape))
    o64 = pl.BlockSpec((BR, Hh), lambda i: (i, 0))
    return pl.pallas_call(
        _prep_body,
        grid=(NBLK,),
        in_specs=[row, row, full(F), full(F), full(F, Hh), full(F, Hh),
                  full(F, H), full(H)],
        out_specs=[o64, o64, o64, o64, row, row],
        out_shape=[jax.ShapeDtypeStruct((NP, Hh), jnp.float32)] * 4
        + [jax.ShapeDtypeStruct((NP, H), jnp.float32)] * 2,
    )(hx, tx, g, b, wnc, wnb, wd, bd)


def _sc_body(y1, y2, y3, y4, srcs, dsts, part,
             sidx, didx, rows, zrow, acc_sh):
    c = lax.axis_index("c")
    s = lax.axis_index("s")
    wid = c * NS + s
    ys = [y1, y2, y3, y4]

    def zero_zrow():
        z = jnp.zeros((16,), jnp.float32)
        for i in range(CK):
            for l in range(Hh // 16):
                zrow[i, pl.ds(l * 16, 16)] = z

    zero_zrow()

    def run():
        for a in range(4):
            pltpu.sync_copy(srcs.at[a, wid], sidx)
            pltpu.sync_copy(dsts.at[a, wid], didx)
            for k in range(ROWS_PER_TILE // CK):
                pltpu.sync_copy(
                    zrow, acc_sh.at[pl.ds(s * ROWS_PER_TILE + k * CK, CK)])
            plsc.subcore_barrier()

            def cbody(j, _):
                pltpu.sync_copy(ys[a].at[sidx.at[j]], rows)
                pltpu.sync_copy(rows, acc_sh.at[didx.at[j]], add=True)
                return 0
            lax.fori_loop(0, CH, cbody, 0)
            plsc.subcore_barrier()
            pltpu.sync_copy(
                acc_sh.at[pl.ds(s * ROWS_PER_TILE, ROWS_PER_TILE)],
                part.at[a, c, pl.ds(s * ROWS_PER_TILE, ROWS_PER_TILE)])
            plsc.subcore_barrier()

    run()


def _sc_agg(y1, y2, y3, y4, srcs, dsts):
    mesh = plsc.VectorSubcoreMesh(core_axis_name="c", subcore_axis_name="s",
                                  num_cores=NC, num_subcores=NS)
    fn = pl.kernel(
        _sc_body,
        out_type=jax.ShapeDtypeStruct((4, NC, NP, Hh), jnp.float32),
        mesh=mesh,
        compiler_params=pltpu.CompilerParams(use_tc_tiling_on_sc=False),
        scratch_types=[
            pltpu.VMEM((CH, CK), jnp.int32),
            pltpu.VMEM((CH, CK), jnp.int32),
            pltpu.VMEM((CK, Hh), jnp.float32),
            pltpu.VMEM((CK, Hh), jnp.float32),
            pltpu.VMEM_SHARED((NP, Hh), jnp.float32),
        ],
    )
    return fn(y1, y2, y3, y4, srcs, dsts)


DEG_EC = 2560
DEG_NJ = E // DEG_EC


def _deg_body(dst_ref, deg_ref):
    j = pl.program_id(1)
    d = dst_ref[0, 0, 0, :]
    hi = lax.shift_right_logical(d, 7)
    lo = jnp.bitwise_and(d, 127)
    oh_hi = (lax.broadcasted_iota(jnp.int32, (NP // 128, DEG_EC), 0)
             == hi[None, :]).astype(jnp.bfloat16)
    oh_lo = (lo[:, None]
             == lax.broadcasted_iota(jnp.int32, (DEG_EC, 128), 1)
             ).astype(jnp.bfloat16)

    @pl.when(j == 0)
    def _():
        deg_ref[...] = jnp.zeros_like(deg_ref)

    deg_ref[0] += jnp.dot(oh_hi, oh_lo, preferred_element_type=jnp.float32)


def _deg(dsts_flat):
    return pl.pallas_call(
        _deg_body,
        grid=(4, DEG_NJ),
        in_specs=[pl.BlockSpec((1, 1, 1, DEG_EC),
                               lambda a, j: (a, j, 0, 0))],
        out_specs=pl.BlockSpec((1, NP // 128, 128), lambda a, j: (a, 0, 0)),
        out_shape=jax.ShapeDtypeStruct((4, NP // 128, 128), jnp.float32),
    )(dsts_flat.reshape(4, DEG_NJ, 1, DEG_EC))


def _combine_body(last, dh_ref, dt_ref, part_ref, deg_ref, hb_ref, tb_ref,
                  g_ref, b_ref, wnc_ref, wnb_ref, wd_ref, bd_ref, *outs):
    if last:
        ph_ref, pt_ref = outs
    else:
        y1_ref, y2_ref, y3_ref, y4_ref, dh1_ref, dt1_ref, ph_ref, pt_ref = outs
    i = pl.program_id(0)
    deg = deg_ref[...]
    invd = 1.0 / jnp.maximum(deg[:, 0, 0], 1.0)
    part = part_ref[...]
    psum = part[:, 0] + part[:, 1]

    h_pre = dh_ref[...] + jnp.concatenate(
        [psum[0] * invd[0][:, None], psum[3] * invd[3][:, None]], axis=1)
    t_pre = dt_ref[...] + jnp.concatenate(
        [psum[1] * invd[1][:, None], psum[2] * invd[2][:, None]], axis=1)

    ids = lax.broadcasted_iota(jnp.int32, (B, BR), 0)
    oh_h = (ids == hb_ref[0, 0][None, :]).astype(jnp.float32)
    oh_t = (ids == tb_ref[0, 0][None, :]).astype(jnp.float32)

    @pl.when(i == 0)
    def _():
        ph_ref[...] = jnp.zeros_like(ph_ref)
        pt_ref[...] = jnp.zeros_like(pt_ref)

    ph_ref[...] += jnp.dot(oh_h, h_pre, preferred_element_type=jnp.float32)
    pt_ref[...] += jnp.dot(oh_t, t_pre, preferred_element_type=jnp.float32)

    if not last:
        g = g_ref[...]
        b = b_ref[...]
        xh = _elu(_elu(_ln(h_pre, g, b)))
        xt = _elu(_elu(_ln(t_pre, g, b)))
        wnc = wnc_ref[...]
        wnb = wnb_ref[...]
        wd = wd_ref[...]
        bd = bd_ref[...]
        y1_ref[...] = jnp.dot(xh, wnc, preferred_element_type=jnp.float32)
        y2_ref[...] = jnp.dot(xt, wnc, preferred_element_type=jnp.float32)
        y3_ref[...] = jnp.dot(xh, wnb, preferred_element_type=jnp.float32)
        y4_ref[...] = jnp.dot(xt, wnb, preferred_element_type=jnp.float32)
        dh1_ref[...] = jnp.dot(xh, wd, preferred_element_type=jnp.float32) + bd
        dt1_ref[...] = jnp.dot(xt, wd, preferred_element_type=jnp.float32) + bd


def _combine(last, dh, dt, part, deg4, hb3, tb3, g, b, wnc, wnb, wd, bd):
    row = pl.BlockSpec((BR, H), lambda i: (i, 0))
    o64 = pl.BlockSpec((BR, Hh), lambda i: (i, 0))
    full = lambda *shape: pl.BlockSpec(shape, lambda i: (0,) * len(shape))
    pool = pl.BlockSpec((B, H), lambda i: (0, 0))
    in_specs = [
        row, row,
        pl.BlockSpec((4, NC, BR, Hh), lambda i: (0, 0, i, 0)),
        pl.BlockSpec((4, 1, 1, BR), lambda i: (0, i, 0, 0)),
        pl.BlockSpec((1, 1, BR), lambda i: (i, 0, 0)),
        pl.BlockSpec((1, 1, BR), lambda i: (i, 0, 0)),
        full(H), full(H), full(H, Hh), full(H, Hh), full(H, H), full(H),
    ]
    if last:
        out_specs = [pool, pool]
        out_shape = [jax.ShapeDtypeStruct((B, H), jnp.float32)] * 2
    else:
        out_specs = [o64, o64, o64, o64, row, row, pool, pool]
        out_shape = ([jax.ShapeDtypeStruct((NP, Hh), jnp.float32)] * 4
                     + [jax.ShapeDtypeStruct((NP, H), jnp.float32)] * 2
                     + [jax.ShapeDtypeStruct((B, H), jnp.float32)] * 2)
    return pl.pallas_call(
        functools.partial(_combine_body, last),
        grid=(NBLK,),
        in_specs=in_specs,
        out_specs=out_specs,
        out_shape=out_shape,
    )(dh, dt, part, deg4, hb3, tb3, g, b, wnc, wnb, wd, bd)


def _score_body(rels_ref, heads_ref, tails_ref, rel_ref, mw_ref, mb_ref, a_ref,
                out_ref):
    del rels_ref
    hh = heads_ref[0]
    tt = tails_ref[0]
    hn = jnp.sqrt(jnp.sum(hh * hh, axis=1, keepdims=True))
    tn = jnp.sqrt(jnp.sum(tt * tt, axis=1, keepdims=True))
    hh = hh / jnp.maximum(hn, 1e-12)
    tt = tt / jnp.maximum(tn, 1e-12)
    m = rel_ref[0]
    fn = jnp.sqrt(jnp.sum(m * m))
    scale = 1.0 / jnp.maximum(fn, 1e-12)
    hm = jnp.dot(hh, m, preferred_element_type=jnp.float32)
    sc = lax.dot_general(hm, tt, (((1,), (1,)), ((), ())),
                         preferred_element_type=jnp.float32)
    sc = sc * scale
    alpha = a_ref[0, 0]
    acc = mb_ref[...]
    for u in range(2):
        for v in range(2):
            val = sc[u, v]
            val = jnp.where(val > 0, val, alpha * val)
            acc = acc + val * mw_ref[u * 2 + v, :]
    out_ref[0, 0, :] = acc


def _score(rels, heads, tails, rel_r, mw, mb, alpha):
    grid_spec = pltpu.PrefetchScalarGridSpec(
        num_scalar_prefetch=1,
        grid=(B,),
        in_specs=[
            pl.BlockSpec((1, 2, H), lambda i, rr: (i, 0, 0)),
            pl.BlockSpec((1, 2, H), lambda i, rr: (i, 0, 0)),
            pl.BlockSpec((1, H, H), lambda i, rr: (rr[i], 0, 0)),
            pl.BlockSpec((4, L), lambda i, rr: (0, 0)),
            pl.BlockSpec((L,), lambda i, rr: (0,)),
            pl.BlockSpec((1, 1), lambda i, rr: (0, 0)),
        ],
        out_specs=pl.BlockSpec((1, 1, L), lambda i, rr: (i, 0, 0)),
    )
    return pl.pallas_call(
        _score_body,
        grid_spec=grid_spec,
        out_shape=jax.ShapeDtypeStruct((B, 1, L), jnp.float32),
    )(rels, heads, tails, rel_r, mw, mb, alpha).reshape(B, L)


def kernel(h_x, h_edge_index, h_batch, t_x, t_edge_index, t_batch, rels,
           b_edge_index, params):
    p = params
    hx = jnp.pad(h_x, ((0, NP - N), (0, 0)))
    tx = jnp.pad(t_x, ((0, NP - N), (0, 0)))
    hb3 = jnp.pad(h_batch, (0, NP - N), constant_values=B).reshape(NBLK, 1, BR)
    tb3 = jnp.pad(t_batch, (0, NP - N), constant_values=B).reshape(NBLK, 1, BR)

    srcs = jnp.stack([h_edge_index[0], t_edge_index[0],
                      b_edge_index[0], b_edge_index[1]]).reshape(4, NW, EP)
    dsts = jnp.stack([h_edge_index[1], t_edge_index[1],
                      b_edge_index[1], b_edge_index[0]]).reshape(4, NW, EP)
    srcs = jnp.pad(srcs, ((0, 0), (0, 0), (0, EPP - EP)),
                   constant_values=0).reshape(4, NW, CH, CK)
    dsts = jnp.pad(dsts, ((0, 0), (0, 0), (0, EPP - EP)),
                   constant_values=N).reshape(4, NW, CH, CK)

    wd = [jnp.concatenate([p[f'conv_Ws_{i}'], p[f'bi_Ws_{i}']], axis=1)
          for i in range(2)]
    bd = [jnp.concatenate([p[f'conv_b_{i}'], p[f'bi_b_{i}']]) for i in range(2)]

    y1, y2, y3, y4, dh, dt = _prep(hx, tx, p['ln0_g'], p['ln0_b'],
                                   p['conv_Wn_0'], p['bi_Wn_0'], wd[0], bd[0])
    dsts_flat = jnp.stack([h_edge_index[1], t_edge_index[1],
                           b_edge_index[1], b_edge_index[0]])
    deg4 = _deg(dsts_flat).reshape(4, NBLK, 1, BR)
    part0 = _sc_agg(y1, y2, y3, y4, srcs, dsts)
    y1b, y2b, y3b, y4b, dh1, dt1, ph0, pt0 = _combine(
        False, dh, dt, part0, deg4, hb3, tb3, p['ln_g_0'], p['ln_b_0'],
        p['conv_Wn_1'], p['bi_Wn_1'], wd[1], bd[1])
    part1 = _sc_agg(y1b, y2b, y3b, y4b, srcs, dsts)
    ph1, pt1 = _combine(True, dh1, dt1, part1, deg4, hb3, tb3,
                        p['ln_g_1'], p['ln_b_1'], p['conv_Wn_1'],
                        p['bi_Wn_1'], wd[1], bd[1])

    heads = jnp.stack([ph0, ph1], axis=1)
    tails = jnp.stack([pt0, pt1], axis=1)
    rel_r = p['rel_emb'].reshape(R, H, H)
    alpha = jnp.reshape(p['prelu_a'], (1, 1))
    return _score(rels, heads, tails, rel_r, p['mlp_W'], p['mlp_b'], alpha)
